# Initial kernel scaffold; baseline (speedup 1.0000x reference)
import jax
import jax.numpy as jnp
from jax.experimental import pallas as pl


def kernel(ids, id_embed, feat_tbl, feat_proj_w, prod_mask, gamma):
    raise NotImplementedError("write your pallas kernel here")


# speedup vs baseline: 7.3574x; 7.3574x over previous
import functools

import jax
import jax.numpy as jnp
from jax import lax
from jax.experimental import pallas as pl
from jax.experimental.pallas import tpu as pltpu
from jax.experimental.pallas import tpu_sc as plsc

VOCAB = 100000
D = 128
ROWS_PER_BLOCK = 800
NC, NS, LANES = 2, 16, 16
NW = NC * NS
CHUNK = 128


def _fuse_body(scale_ref, emb_ref, feat_ref, w_ref, out_ref):
    proj = lax.dot_general(
        feat_ref[...], w_ref[...],
        dimension_numbers=(((1,), (1,)), ((), ())),
        preferred_element_type=jnp.float32,
    )
    out_ref[...] = emb_ref[...] + scale_ref[...] * proj


def _fused_table(scale, id_embed, feat_tbl, feat_proj_w):
    grid = VOCAB // ROWS_PER_BLOCK
    return pl.pallas_call(
        _fuse_body,
        grid=(grid,),
        in_specs=[
            pl.BlockSpec((ROWS_PER_BLOCK, 1), lambda i: (i, 0)),
            pl.BlockSpec((ROWS_PER_BLOCK, D), lambda i: (i, 0)),
            pl.BlockSpec((ROWS_PER_BLOCK, D), lambda i: (i, 0)),
            pl.BlockSpec((D, D), lambda i: (0, 0)),
        ],
        out_specs=pl.BlockSpec((ROWS_PER_BLOCK, D), lambda i: (i, 0)),
        out_shape=jax.ShapeDtypeStruct((VOCAB, D), jnp.float32),
    )(scale, id_embed, feat_tbl, feat_proj_w)


def _gather_body(n_chunks, tbl_hbm, idx_hbm, out_hbm, idx_v, rows_a, rows_b,
                 sem_a, sem_b):
    wid = lax.axis_index("s") * NC + lax.axis_index("c")
    tok0 = wid * n_chunks * CHUNK
    pltpu.sync_copy(idx_hbm.at[pl.ds(tok0, n_chunks * CHUNK)], idx_v)

    def start(j, buf, sem):
        return pltpu.async_copy(tbl_hbm.at[idx_v.at[pl.ds(j * CHUNK, CHUNK)]],
                                buf, sem)

    def store(j, buf):
        pltpu.sync_copy(buf, out_hbm.at[pl.ds(tok0 + j * CHUNK, CHUNK)])

    @pl.loop(0, n_chunks // 2)
    def _(p):
        j = 2 * p
        h0 = start(j, rows_a, sem_a)
        h1 = start(j + 1, rows_b, sem_b)
        h0.wait()
        store(j, rows_a)
        h1.wait()
        store(j + 1, rows_b)


def _sc_gather(fused, idx_flat, n_tokens):
    n_chunks = n_tokens // (NW * CHUNK)
    mesh = plsc.VectorSubcoreMesh(core_axis_name="c", subcore_axis_name="s")
    return pl.kernel(
        functools.partial(_gather_body, n_chunks),
        out_type=jax.ShapeDtypeStruct((n_tokens, D), jnp.float32),
        mesh=mesh,
        scratch_types=[
            pltpu.VMEM((n_chunks * CHUNK,), jnp.int32),
            pltpu.VMEM((CHUNK, D), jnp.float32),
            pltpu.VMEM((CHUNK, D), jnp.float32),
            pltpu.SemaphoreType.DMA,
            pltpu.SemaphoreType.DMA,
        ],
    )(fused, idx_flat)


def kernel(ids, id_embed, feat_tbl, feat_proj_w, prod_mask, gamma):
    B, L = ids.shape
    n_tokens = B * L
    scale = (gamma * prod_mask.astype(jnp.float32)).reshape(VOCAB, 1)
    fused = _fused_table(scale, id_embed, feat_tbl, feat_proj_w)
    out = _sc_gather(fused, ids.reshape(n_tokens), n_tokens)
    return out.reshape(B, L, D)


# speedup vs baseline: 10.4046x; 1.4142x over previous
import functools

import jax
import jax.numpy as jnp
from jax import lax
from jax.experimental import pallas as pl
from jax.experimental.pallas import tpu as pltpu
from jax.experimental.pallas import tpu_sc as plsc

VOCAB = 100000
D = 128
ROWS_PER_BLOCK = 800
NC, NS, LANES = 2, 16, 16
NW = NC * NS
CHUNK = 128


def _fuse_body(scale_ref, emb_ref, feat_ref, w_ref, out_ref):
    proj = lax.dot_general(
        feat_ref[...], w_ref[...],
        dimension_numbers=(((1,), (1,)), ((), ())),
        preferred_element_type=jnp.float32,
    )
    out_ref[...] = emb_ref[...] + scale_ref[...] * proj


def _fused_table(scale, id_embed, feat_tbl, feat_proj_w):
    grid = VOCAB // ROWS_PER_BLOCK
    return pl.pallas_call(
        _fuse_body,
        grid=(grid,),
        in_specs=[
            pl.BlockSpec((ROWS_PER_BLOCK, 1), lambda i: (i, 0)),
            pl.BlockSpec((ROWS_PER_BLOCK, D), lambda i: (i, 0)),
            pl.BlockSpec((ROWS_PER_BLOCK, D), lambda i: (i, 0)),
            pl.BlockSpec((D, D), lambda i: (0, 0)),
        ],
        out_specs=pl.BlockSpec((ROWS_PER_BLOCK, D), lambda i: (i, 0)),
        out_shape=jax.ShapeDtypeStruct((VOCAB, D), jnp.float32),
    )(scale, id_embed, feat_tbl, feat_proj_w)


def _gather_body(n_rows, seq, group, tbl_hbm, idx_hbm, out_hbm, idx_v, buf_a,
                 buf_b, sem_a, sem_b):
    wid = lax.axis_index("s") * NC + lax.axis_index("c")
    r0 = wid * n_rows
    pltpu.sync_copy(idx_hbm.at[pl.ds(r0, n_rows)], idx_v)

    def fire(g, buf, sem):
        return [
            pltpu.async_copy(tbl_hbm.at[idx_v.at[g * group + j]], buf.at[j],
                             sem)
            for j in range(group)
        ]

    def drain_store(g, hs, buf):
        for h in hs:
            h.wait()
        pltpu.sync_copy(buf, out_hbm.at[pl.ds(r0 + g * group, group)])

    @pl.loop(0, n_rows // group // 2)
    def _(p):
        g = 2 * p
        ha = fire(g, buf_a, sem_a)
        hb = fire(g + 1, buf_b, sem_b)
        drain_store(g, ha, buf_a)
        drain_store(g + 1, hb, buf_b)


def _sc_gather(fused, ids):
    n_seq, seq = ids.shape
    n_rows = n_seq // NW
    group = 8
    mesh = plsc.VectorSubcoreMesh(core_axis_name="c", subcore_axis_name="s")
    return pl.kernel(
        functools.partial(_gather_body, n_rows, seq, group),
        out_type=jax.ShapeDtypeStruct((n_seq, seq, D), jnp.float32),
        mesh=mesh,
        scratch_types=[
            pltpu.VMEM((n_rows, seq), jnp.int32),
            pltpu.VMEM((group, seq, D), jnp.float32),
            pltpu.VMEM((group, seq, D), jnp.float32),
            pltpu.SemaphoreType.DMA,
            pltpu.SemaphoreType.DMA,
        ],
    )(fused, ids)


def kernel(ids, id_embed, feat_tbl, feat_proj_w, prod_mask, gamma):
    scale = (gamma * prod_mask.astype(jnp.float32)).reshape(VOCAB, 1)
    fused = _fused_table(scale, id_embed, feat_tbl, feat_proj_w)
    return _sc_gather(fused, ids)
